# baseline (speedup 1.0000x reference)
import jax
import jax.numpy as jnp
from jax.experimental import pallas as pl
from jax.experimental.pallas import tpu as pltpu

_TM = 1024


def _fused_body(x_ref, lab_ref, w_ref, b_ref, o_ref, lab_o_ref):
    o_ref[...] = (
        jnp.dot(x_ref[...], w_ref[...], preferred_element_type=jnp.float32)
        + b_ref[...]
    )
    lab_o_ref[...] = lab_ref[...].astype(jnp.float32)


def _mlp_body(x_ref, w_ref, b_ref, o_ref):
    o_ref[...] = (
        jnp.dot(x_ref[...], w_ref[...], preferred_element_type=jnp.float32)
        + b_ref[...]
    )


def kernel(pooled_features, labels, weight, bias):
    b, g, d = pooled_features.shape
    out = weight.shape[1]
    n = b * g

    flat = pooled_features.reshape(n, d).astype(jnp.float32)
    w = weight.astype(jnp.float32)
    bias2d = bias.astype(jnp.float32).reshape(1, out)

    tm = min(_TM, n)
    if n % tm == 0 and tm % g == 0:
        grid = (n // tm,)
        feats, label = pl.pallas_call(
            _fused_body,
            out_shape=(
                jax.ShapeDtypeStruct((n, out), jnp.float32),
                jax.ShapeDtypeStruct((b, g), jnp.float32),
            ),
            grid=grid,
            in_specs=[
                pl.BlockSpec((tm, d), lambda i: (i, 0)),
                pl.BlockSpec((tm // g, g), lambda i: (i, 0)),
                pl.BlockSpec((d, out), lambda i: (0, 0)),
                pl.BlockSpec((1, out), lambda i: (0, 0)),
            ],
            out_specs=(
                pl.BlockSpec((tm, out), lambda i: (i, 0)),
                pl.BlockSpec((tm // g, g), lambda i: (i, 0)),
            ),
            compiler_params=pltpu.CompilerParams(
                dimension_semantics=("parallel",),
            ),
        )(flat, labels, w, bias2d)
        return feats, label

    grid = (pl.cdiv(n, tm),)
    feats = pl.pallas_call(
        _mlp_body,
        out_shape=jax.ShapeDtypeStruct((n, out), jnp.float32),
        grid=grid,
        in_specs=[
            pl.BlockSpec((tm, d), lambda i: (i, 0)),
            pl.BlockSpec((d, out), lambda i: (0, 0)),
            pl.BlockSpec((1, out), lambda i: (0, 0)),
        ],
        out_specs=pl.BlockSpec((tm, out), lambda i: (i, 0)),
        compiler_params=pltpu.CompilerParams(
            dimension_semantics=("parallel",),
        ),
    )(flat, w, bias2d)
    return feats, labels.astype(jnp.float32)


# speedup vs baseline: 1.1115x; 1.1115x over previous
import jax
import jax.numpy as jnp
from jax.experimental import pallas as pl
from jax.experimental.pallas import tpu as pltpu

_TM = 2048


def _fused_body(x_ref, lab_ref, w_ref, b_ref, o_ref, lab_o_ref):
    o_ref[...] = (
        jnp.dot(x_ref[...], w_ref[...], preferred_element_type=jnp.float32)
        + b_ref[...]
    )
    lab_o_ref[...] = lab_ref[...].astype(jnp.float32)


def _mlp_body(x_ref, w_ref, b_ref, o_ref):
    o_ref[...] = (
        jnp.dot(x_ref[...], w_ref[...], preferred_element_type=jnp.float32)
        + b_ref[...]
    )


def kernel(pooled_features, labels, weight, bias):
    b, g, d = pooled_features.shape
    out = weight.shape[1]
    n = b * g

    flat = pooled_features.reshape(n, d).astype(jnp.float32)
    w = weight.astype(jnp.float32)
    bias2d = bias.astype(jnp.float32).reshape(1, out)

    tm = min(_TM, n)
    if n % tm == 0 and tm % g == 0:
        grid = (n // tm,)
        feats, label = pl.pallas_call(
            _fused_body,
            out_shape=(
                jax.ShapeDtypeStruct((n, out), jnp.float32),
                jax.ShapeDtypeStruct((b, g), jnp.float32),
            ),
            grid=grid,
            in_specs=[
                pl.BlockSpec((tm, d), lambda i: (i, 0)),
                pl.BlockSpec((tm // g, g), lambda i: (i, 0)),
                pl.BlockSpec((d, out), lambda i: (0, 0)),
                pl.BlockSpec((1, out), lambda i: (0, 0)),
            ],
            out_specs=(
                pl.BlockSpec((tm, out), lambda i: (i, 0)),
                pl.BlockSpec((tm // g, g), lambda i: (i, 0)),
            ),
            compiler_params=pltpu.CompilerParams(
                dimension_semantics=("parallel",),
            ),
        )(flat, labels, w, bias2d)
        return feats, label

    grid = (pl.cdiv(n, tm),)
    feats = pl.pallas_call(
        _mlp_body,
        out_shape=jax.ShapeDtypeStruct((n, out), jnp.float32),
        grid=grid,
        in_specs=[
            pl.BlockSpec((tm, d), lambda i: (i, 0)),
            pl.BlockSpec((d, out), lambda i: (0, 0)),
            pl.BlockSpec((1, out), lambda i: (0, 0)),
        ],
        out_specs=pl.BlockSpec((tm, out), lambda i: (i, 0)),
        compiler_params=pltpu.CompilerParams(
            dimension_semantics=("parallel",),
        ),
    )(flat, w, bias2d)
    return feats, labels.astype(jnp.float32)
